# Initial kernel scaffold; baseline (speedup 1.0000x reference)
import jax
import jax.numpy as jnp
from jax.experimental import pallas as pl


def kernel(sentences_hidden, sentences_num, sentences_mask, sent_adjacent_matrix, head_type, edge_type, node_query, W_hp, b_hp, W_ql, b_ql, W_kl, b_kl, g_q, beta_q, g_k, beta_k, flag_embed, edge_embed, Wq, Wk, Wv, We):
    raise NotImplementedError("write your pallas kernel here")


# speedup vs baseline: 3.9821x; 3.9821x over previous
import functools

import jax
import jax.numpy as jnp
from jax.experimental import pallas as pl

D = 512
_INV_SQRT_D = 1.0 / (512.0 ** 0.5)


def _fused_kernel(
    sh_ref,
    mask_ref,
    adj_ref,
    ht_ref,
    et_ref,
    nq_ref,
    whp_ref,
    bhp_ref,
    wql_ref,
    bql_ref,
    wkl_ref,
    bkl_ref,
    gq_ref,
    betaq_ref,
    gk_ref,
    betak_ref,
    flag_ref,
    eemb_ref,
    wq_ref,
    wk_ref,
    wv_ref,
    we_ref,
    hidden_ref,
    recall_ref,
):
    f32 = jnp.float32
    s = sh_ref[0]
    m = mask_ref[0]

    sl = m.sum(axis=1, keepdims=True)
    sl_safe = jnp.where(sl != 0.0, sl, 1.0)
    pooled = (s * m[:, :, None]).sum(axis=1) / sl_safe

    node = jnp.dot(pooled, whp_ref[...], preferred_element_type=f32) + bhp_ref[...]

    ht = ht_ref[0]
    h = node + jnp.where(ht == 1, flag_ref[1:2, :], flag_ref[0:1, :])

    q = jnp.dot(h, wq_ref[...], preferred_element_type=f32)
    k = jnp.dot(h, wk_ref[...], preferred_element_type=f32)
    v = jnp.dot(h, wv_ref[...], preferred_element_type=f32)

    e_proj = jnp.dot(eemb_ref[...], we_ref[...], preferred_element_type=f32)
    qE = jax.lax.dot_general(q, e_proj, (((1,), (1,)), ((), ())),
                             preferred_element_type=f32)

    et = et_ref[0]
    escore = jnp.zeros(et.shape, dtype=f32)
    for t in range(5):
        escore = jnp.where(et == t, qE[:, t:t + 1], escore)

    qk = jax.lax.dot_general(q, k, (((1,), (1,)), ((), ())),
                             preferred_element_type=f32)
    scores = (qk + escore) * _INV_SQRT_D

    adj = adj_ref[0]
    scores = jnp.where(adj > 0, scores, -1e9)
    mx = scores.max(axis=1, keepdims=True)
    p = jnp.exp(scores - mx)
    attn = p / p.sum(axis=1, keepdims=True)
    row_has = (adj.sum(axis=1, keepdims=True) > 0).astype(f32)
    attn = attn * row_has

    hidden = jnp.dot(attn, v, preferred_element_type=f32) + h
    hidden_ref[0] = hidden

    def _ln(x, g, b):
        mu = x.mean(axis=1, keepdims=True)
        var = ((x - mu) ** 2).mean(axis=1, keepdims=True)
        return (x - mu) / jnp.sqrt(var + 1e-5) * g + b

    key = _ln(jnp.dot(hidden, wkl_ref[...], preferred_element_type=f32)
              + bkl_ref[...], gk_ref[...], betak_ref[...])
    nq = nq_ref[0]
    qry = _ln(jnp.dot(nq, wql_ref[...], preferred_element_type=f32)
              + bql_ref[...], gq_ref[...], betaq_ref[...])

    logits = (key * qry).sum(axis=1, keepdims=True)
    pad = (sl != 0.0).astype(f32)
    recall_ref[0] = jax.nn.sigmoid(logits) * pad


@jax.jit
def kernel(sentences_hidden, sentences_num, sentences_mask, sent_adjacent_matrix,
           head_type, edge_type, node_query, W_hp, b_hp, W_ql, b_ql, W_kl, b_kl,
           g_q, beta_q, g_k, beta_k, flag_embed, edge_embed, Wq, Wk, Wv, We):
    B = sentences_num.shape[0]
    BS, L, DH = sentences_hidden.shape
    S = BS // B

    sh4 = sentences_hidden.reshape(B, S, L, DH)
    mask3 = sentences_mask.reshape(B, S, L)
    adj = sent_adjacent_matrix.astype(jnp.int32)
    ht3 = head_type.astype(jnp.int32).reshape(B, S, 1)
    et3 = edge_type.astype(jnp.int32)
    nq3 = node_query.reshape(B, 1, DH)
    eemb8 = jnp.zeros((8, D), jnp.float32).at[:5].set(edge_embed)

    row2 = lambda x: x.reshape(1, D)

    grid = (B,)
    data_spec = lambda rank: pl.BlockSpec(
        (1,) + rank, lambda b: (b,) + (0,) * len(rank))
    full_spec = lambda shp: pl.BlockSpec(shp, lambda b: (0,) * len(shp))

    hidden, recall = pl.pallas_call(
        _fused_kernel,
        grid=grid,
        in_specs=[
            data_spec((S, L, DH)),
            data_spec((S, L)),
            data_spec((S, S)),
            data_spec((S, 1)),
            data_spec((S, S)),
            data_spec((1, DH)),
            full_spec((DH, D)),
            full_spec((1, D)),
            full_spec((DH, D)),
            full_spec((1, D)),
            full_spec((D, D)),
            full_spec((1, D)),
            full_spec((1, D)),
            full_spec((1, D)),
            full_spec((1, D)),
            full_spec((1, D)),
            full_spec((2, D)),
            full_spec((8, D)),
            full_spec((D, D)),
            full_spec((D, D)),
            full_spec((D, D)),
            full_spec((D, D)),
        ],
        out_specs=[
            data_spec((S, D)),
            data_spec((S, 1)),
        ],
        out_shape=[
            jax.ShapeDtypeStruct((B, S, D), jnp.float32),
            jax.ShapeDtypeStruct((B, S, 1), jnp.float32),
        ],
    )(sh4, mask3, adj, ht3, et3, nq3,
      W_hp, row2(b_hp), W_ql, row2(b_ql), W_kl, row2(b_kl),
      row2(g_q), row2(beta_q), row2(g_k), row2(beta_k),
      flag_embed, eemb8, Wq, Wk, Wv, We)

    return recall.reshape(B, S), hidden


# speedup vs baseline: 4.7313x; 1.1881x over previous
import functools

import jax
import jax.numpy as jnp
from jax.experimental import pallas as pl

D = 512
NB = 2
_INV_SQRT_D = 1.0 / (512.0 ** 0.5)


def _fused_kernel(
    sh_ref,
    mask_ref,
    adj_ref,
    ht_ref,
    et_ref,
    nq_ref,
    whp_ref,
    bhp_ref,
    wql_ref,
    bql_ref,
    wkl_ref,
    bkl_ref,
    gq_ref,
    betaq_ref,
    gk_ref,
    betak_ref,
    flag_ref,
    eemb_ref,
    wq_ref,
    wk_ref,
    wv_ref,
    we_ref,
    hidden_ref,
    recall_ref,
):
    f32 = jnp.float32
    nb, S, L, DH = sh_ref.shape
    R = nb * S
    s = sh_ref[...].reshape(R, L, DH)
    m = mask_ref[...].reshape(R, L)

    sl = m.sum(axis=1, keepdims=True)
    sl_safe = jnp.where(sl != 0.0, sl, 1.0)
    pooled = (s * m[:, :, None]).sum(axis=1) / sl_safe

    node = jnp.dot(pooled, whp_ref[...], preferred_element_type=f32) + bhp_ref[...]

    ht = ht_ref[...].reshape(R, 1)
    h = node + jnp.where(ht == 1, flag_ref[1:2, :], flag_ref[0:1, :])

    q = jnp.dot(h, wq_ref[...], preferred_element_type=f32)
    k = jnp.dot(h, wk_ref[...], preferred_element_type=f32)
    v = jnp.dot(h, wv_ref[...], preferred_element_type=f32)

    e_proj = jnp.dot(eemb_ref[...], we_ref[...], preferred_element_type=f32)
    qE = jax.lax.dot_general(q, e_proj, (((1,), (1,)), ((), ())),
                             preferred_element_type=f32)

    def _ln(x, g, b):
        mu = x.mean(axis=1, keepdims=True)
        var = ((x - mu) ** 2).mean(axis=1, keepdims=True)
        return (x - mu) / jnp.sqrt(var + 1e-5) * g + b

    nq = nq_ref[...].reshape(nb, DH)
    qry = _ln(jnp.dot(nq, wql_ref[...], preferred_element_type=f32)
              + bql_ref[...], gq_ref[...], betaq_ref[...])

    for b in range(nb):
        r0 = b * S
        qb = q[r0:r0 + S]
        kb = k[r0:r0 + S]
        vb = v[r0:r0 + S]
        hb = h[r0:r0 + S]

        et = et_ref[b]
        escore = jnp.zeros(et.shape, dtype=f32)
        for t in range(5):
            escore = jnp.where(et == t, qE[r0:r0 + S, t:t + 1], escore)

        qk = jax.lax.dot_general(qb, kb, (((1,), (1,)), ((), ())),
                                 preferred_element_type=f32)
        scores = (qk + escore) * _INV_SQRT_D

        adj = adj_ref[b]
        scores = jnp.where(adj > 0, scores, -1e9)
        mx = scores.max(axis=1, keepdims=True)
        p = jnp.exp(scores - mx)
        attn = p / p.sum(axis=1, keepdims=True)
        row_has = (adj.sum(axis=1, keepdims=True) > 0).astype(f32)
        attn = attn * row_has

        hidden = jnp.dot(attn, vb, preferred_element_type=f32) + hb
        hidden_ref[b] = hidden

        key = _ln(jnp.dot(hidden, wkl_ref[...], preferred_element_type=f32)
                  + bkl_ref[...], gk_ref[...], betak_ref[...])
        logits = (key * qry[b:b + 1]).sum(axis=1, keepdims=True)
        pad = (sl[r0:r0 + S] != 0.0).astype(f32)
        recall_ref[b] = jax.nn.sigmoid(logits) * pad


@jax.jit
def kernel(sentences_hidden, sentences_num, sentences_mask, sent_adjacent_matrix,
           head_type, edge_type, node_query, W_hp, b_hp, W_ql, b_ql, W_kl, b_kl,
           g_q, beta_q, g_k, beta_k, flag_embed, edge_embed, Wq, Wk, Wv, We):
    B = sentences_num.shape[0]
    BS, L, DH = sentences_hidden.shape
    S = BS // B

    sh4 = sentences_hidden.reshape(B, S, L, DH)
    mask3 = sentences_mask.reshape(B, S, L)
    adj = sent_adjacent_matrix.astype(jnp.int32)
    ht3 = head_type.astype(jnp.int32).reshape(B, S, 1)
    et3 = edge_type.astype(jnp.int32)
    nq3 = node_query.reshape(B, 1, DH)
    eemb8 = jnp.zeros((8, D), jnp.float32).at[:5].set(edge_embed)

    row2 = lambda x: x.reshape(1, D)

    grid = (B // NB,)
    data_spec = lambda rank: pl.BlockSpec(
        (NB,) + rank, lambda b: (b,) + (0,) * len(rank))
    full_spec = lambda shp: pl.BlockSpec(shp, lambda b: (0,) * len(shp))

    hidden, recall = pl.pallas_call(
        _fused_kernel,
        grid=grid,
        in_specs=[
            data_spec((S, L, DH)),
            data_spec((S, L)),
            data_spec((S, S)),
            data_spec((S, 1)),
            data_spec((S, S)),
            data_spec((1, DH)),
            full_spec((DH, D)),
            full_spec((1, D)),
            full_spec((DH, D)),
            full_spec((1, D)),
            full_spec((D, D)),
            full_spec((1, D)),
            full_spec((1, D)),
            full_spec((1, D)),
            full_spec((1, D)),
            full_spec((1, D)),
            full_spec((2, D)),
            full_spec((8, D)),
            full_spec((D, D)),
            full_spec((D, D)),
            full_spec((D, D)),
            full_spec((D, D)),
        ],
        out_specs=[
            data_spec((S, D)),
            data_spec((S, 1)),
        ],
        out_shape=[
            jax.ShapeDtypeStruct((B, S, D), jnp.float32),
            jax.ShapeDtypeStruct((B, S, 1), jnp.float32),
        ],
    )(sh4, mask3, adj, ht3, et3, nq3,
      W_hp, row2(b_hp), W_ql, row2(b_ql), W_kl, row2(b_kl),
      row2(g_q), row2(beta_q), row2(g_k), row2(beta_k),
      flag_embed, eemb8, Wq, Wk, Wv, We)

    return recall.reshape(B, S), hidden
